# Initial kernel scaffold; baseline (speedup 1.0000x reference)
import jax
import jax.numpy as jnp
from jax.experimental import pallas as pl


def kernel(node_embeddings, triples, basis, att, bias):
    raise NotImplementedError("write your pallas kernel here")


# speedup vs baseline: 10.6743x; 10.6743x over previous
import functools

import jax
import jax.numpy as jnp
from jax import lax
from jax.experimental import pallas as pl
from jax.experimental.pallas import tpu as pltpu
from jax.experimental.pallas import tpu_sc as plsc

_N = 10000
_R = 16
_B = 8
_D = 128
_E = 320000

_NS = 16

_CH = 80
_W_EDGES = _E // _NS
_W_ROWS = _W_EDGES // _CH
_SB = 25
_NSB = _W_ROWS // _SB
_RN = _R * _N
_NPAD = 10112
_NPT = _NPAD // _NS
_ZR = 8
_ZD = 1000


def _h_body(att_ref, basis_ref, emb_ref, h_ref):
    r = pl.program_id(0)
    w = att_ref[r, 0] * basis_ref[0]
    for b in range(1, _B):
        w = w + att_ref[r, b] * basis_ref[b]
    h_ref[0] = jnp.dot(emb_ref[...], w, preferred_element_type=jnp.float32)


def _build_h(emb, basis, att):
    return pl.pallas_call(
        _h_body,
        grid=(_R,),
        in_specs=[
            pl.BlockSpec((_R, _B), lambda r: (0, 0)),
            pl.BlockSpec((_B, _D, _D), lambda r: (0, 0, 0)),
            pl.BlockSpec((_N, _D), lambda r: (0, 0)),
        ],
        out_specs=pl.BlockSpec((1, _N, _D), lambda r: (r, 0, 0)),
        out_shape=jax.ShapeDtypeStruct((_R, _N, _D), jnp.float32),
    )(att, basis, emb)


_mesh = plsc.VectorSubcoreMesh(core_axis_name="c", subcore_axis_name="s",
                               num_cores=1)


@functools.partial(
    pl.kernel,
    out_type=jax.ShapeDtypeStruct((_NS, _NPT, _D), jnp.float32),
    mesh=_mesh,
    scratch_types=[
        pltpu.VMEM((_SB, _CH), jnp.int32),
        pltpu.VMEM((_SB, _CH), jnp.int32),
        pltpu.VMEM((_SB, _CH), jnp.int32),
        pltpu.VMEM((_CH,), jnp.float32),
        pltpu.VMEM((_CH,), jnp.float32),
        pltpu.VMEM((_ZD,), jnp.float32),
        pltpu.VMEM((_ZR, _D), jnp.float32),
        pltpu.VMEM((_CH, _D), jnp.float32),
        pltpu.VMEM_SHARED((_RN,), jnp.float32),
        pltpu.VMEM_SHARED((_NPAD, _D), jnp.float32),
        pltpu.SemaphoreType.DMA,
    ],
)
def _sc_aggregate(h_hbm, gidx_hbm, didx_hbm, sidx_hbm, out_hbm,
                  gblk_v, sblk_v, dblk_v, dtmp_v, ones_v,
                  zdeg_v, zrow_v, rows_v, deg_s, out_s, sem):
    s = lax.axis_index("s")

    zero16 = jnp.zeros((16,), jnp.float32)
    one16 = jnp.ones((16,), jnp.float32)
    for k in range(_ZD // 16):
        zdeg_v[pl.ds(k * 16, 16)] = zero16
    for r in range(_ZR):
        for k in range(_D // 16):
            zrow_v[r, pl.ds(k * 16, 16)] = zero16
    for k in range(_CH // 16):
        ones_v[pl.ds(k * 16, 16)] = one16

    for i in range(_RN // _NS // _ZD):
        pltpu.sync_copy(zdeg_v, deg_s.at[pl.ds(s * (_RN // _NS) + i * _ZD, _ZD)])
    for i in range(_NPT // _ZR):
        pltpu.sync_copy(zrow_v, out_s.at[pl.ds(s * _NPT + i * _ZR, _ZR)])
    plsc.subcore_barrier()

    def deg_blk(sb, carry):
        pltpu.sync_copy(didx_hbm.at[s, sb], dblk_v)

        def deg_body(j, carry2):
            pltpu.sync_copy(ones_v, deg_s.at[dblk_v.at[j]], add=True)
            return carry2
        lax.fori_loop(0, _SB, deg_body, 0)
        return carry
    lax.fori_loop(0, _NSB, deg_blk, 0)
    plsc.subcore_barrier()

    def main_blk(sb, carry):
        pltpu.sync_copy(gidx_hbm.at[s, sb], gblk_v)
        pltpu.sync_copy(sidx_hbm.at[s, sb], sblk_v)
        pltpu.sync_copy(didx_hbm.at[s, sb], dblk_v)

        def main_body(j, carry2):
            pltpu.async_copy(h_hbm.at[gblk_v.at[j]], rows_v, sem).wait()
            pltpu.sync_copy(deg_s.at[dblk_v.at[j]], dtmp_v)

            def grp_body(g, carry3):
                cvec = 1.0 / dtmp_v[pl.ds(g * 16, 16)]
                for el in range(16):
                    cb = jnp.full((16,), cvec[el], jnp.float32)
                    e = g * 16 + el
                    for k in range(_D // 16):
                        rows_v[e, pl.ds(k * 16, 16)] = (
                            rows_v[e, pl.ds(k * 16, 16)] * cb)
                return carry3
            lax.fori_loop(0, _CH // 16, grp_body, 0)

            pltpu.sync_copy(rows_v, out_s.at[sblk_v.at[j]], add=True)
            return carry2
        lax.fori_loop(0, _SB, main_body, 0)
        return carry
    lax.fori_loop(0, _NSB, main_blk, 0)
    plsc.subcore_barrier()

    pltpu.sync_copy(out_s.at[pl.ds(s * _NPT, _NPT)], out_hbm.at[s])


def _comb_body(p_ref, b_ref, o_ref):
    o_ref[...] = p_ref[:_N] + b_ref[...]


def _combine(partial, bias2d):
    return pl.pallas_call(
        _comb_body,
        out_shape=jax.ShapeDtypeStruct((_N, _D), jnp.float32),
    )(partial, bias2d)


def kernel(node_embeddings, triples, basis, att, bias):
    src = triples[:, 0].astype(jnp.int32)
    rel = triples[:, 1].astype(jnp.int32)
    dst = triples[:, 2].astype(jnp.int32)
    gidx = (rel * _N + dst).reshape(_NS, _NSB, _SB, _CH)
    didx = (rel * _N + src).reshape(_NS, _NSB, _SB, _CH)
    sidx = src.reshape(_NS, _NSB, _SB, _CH)
    h = _build_h(node_embeddings, basis, att).reshape(_RN, _D)
    partial = _sc_aggregate(h, gidx, didx, sidx)
    partial = partial.reshape(_NPAD, _D)
    return _combine(partial, bias.reshape(1, _D))


# speedup vs baseline: 15.0259x; 1.4077x over previous
import functools

import jax
import jax.numpy as jnp
from jax import lax
from jax.experimental import pallas as pl
from jax.experimental.pallas import tpu as pltpu
from jax.experimental.pallas import tpu_sc as plsc

_N = 10000
_R = 16
_B = 8
_D = 128
_E = 320000

_NS = 16

_CH = 80
_W_EDGES = _E // _NS
_W_ROWS = _W_EDGES // _CH
_SB = 25
_NSB = _W_ROWS // _SB
_RN = _R * _N
_NPAD = 10112
_NPT = _NPAD // _NS
_ZR = 8
_ZD = 1000


def _h_body(att_ref, basis_ref, emb_ref, h_ref):
    r = pl.program_id(0)
    w = att_ref[r, 0] * basis_ref[0]
    for b in range(1, _B):
        w = w + att_ref[r, b] * basis_ref[b]
    h_ref[0] = jnp.dot(emb_ref[...], w, preferred_element_type=jnp.float32)


def _build_h(emb, basis, att):
    return pl.pallas_call(
        _h_body,
        grid=(_R,),
        in_specs=[
            pl.BlockSpec((_R, _B), lambda r: (0, 0)),
            pl.BlockSpec((_B, _D, _D), lambda r: (0, 0, 0)),
            pl.BlockSpec((_N, _D), lambda r: (0, 0)),
        ],
        out_specs=pl.BlockSpec((1, _N, _D), lambda r: (r, 0, 0)),
        out_shape=jax.ShapeDtypeStruct((_R, _N, _D), jnp.float32),
    )(att, basis, emb)


_mesh = plsc.VectorSubcoreMesh(core_axis_name="c", subcore_axis_name="s",
                               num_cores=1)


@functools.partial(
    pl.kernel,
    out_type=jax.ShapeDtypeStruct((_NS, _NSB, _SB, _CH), jnp.float32),
    mesh=_mesh,
    scratch_types=[
        pltpu.VMEM((_SB, _CH), jnp.int32),
        pltpu.VMEM((_SB, _CH), jnp.float32),
        pltpu.VMEM((_CH,), jnp.float32),
        pltpu.VMEM((_CH,), jnp.float32),
        pltpu.VMEM((_ZD,), jnp.float32),
        pltpu.VMEM_SHARED((_RN,), jnp.float32),
        pltpu.SemaphoreType.DMA,
    ],
)
def _sc_degc(didx_hbm, c_hbm, dblk_v, cblk_v, dtmp_v, ones_v, zdeg_v,
             deg_s, sem_d):
    s = lax.axis_index("s")

    zero16 = jnp.zeros((16,), jnp.float32)
    one16 = jnp.ones((16,), jnp.float32)
    for k in range(_ZD // 16):
        zdeg_v[pl.ds(k * 16, 16)] = zero16
    for k in range(_CH // 16):
        ones_v[pl.ds(k * 16, 16)] = one16

    for i in range(_RN // _NS // _ZD):
        pltpu.sync_copy(zdeg_v, deg_s.at[pl.ds(s * (_RN // _NS) + i * _ZD, _ZD)])
    plsc.subcore_barrier()

    def deg_blk(sb, carry):
        pltpu.sync_copy(didx_hbm.at[s, sb], dblk_v)

        def deg_body(j, carry2):
            pltpu.sync_copy(ones_v, deg_s.at[dblk_v.at[j]], add=True)
            return carry2
        lax.fori_loop(0, _SB, deg_body, 0)
        return carry
    lax.fori_loop(0, _NSB, deg_blk, 0)
    plsc.subcore_barrier()

    def c_blk(sb, carry):
        pltpu.sync_copy(didx_hbm.at[s, sb], dblk_v)

        def c_body(j, carry2):
            pltpu.sync_copy(deg_s.at[dblk_v.at[j]], dtmp_v)
            for k in range(_CH // 16):
                cblk_v[j, pl.ds(k * 16, 16)] = 1.0 / dtmp_v[pl.ds(k * 16, 16)]
            return carry2
        lax.fori_loop(0, _SB, c_body, 0)
        pltpu.sync_copy(cblk_v, c_hbm.at[s, sb])
        return carry
    lax.fori_loop(0, _NSB, c_blk, 0)


@functools.partial(
    pl.kernel,
    out_type=jax.ShapeDtypeStruct((_NS, _NPT, _D), jnp.float32),
    mesh=_mesh,
    scratch_types=[
        pltpu.VMEM((_SB, _CH), jnp.int32),
        pltpu.VMEM((_SB, _CH), jnp.int32),
        pltpu.VMEM((_SB, _CH), jnp.float32),
        pltpu.VMEM((_ZR, _D), jnp.float32),
        pltpu.VMEM((2, _CH, _D), jnp.float32),
        pltpu.VMEM_SHARED((_NPAD, _D), jnp.float32),
        pltpu.SemaphoreType.DMA,
    ],
)
def _sc_aggregate(h_hbm, gidx_hbm, sidx_hbm, c_hbm, out_hbm,
                  gblk_v, sblk_v, cblk_v, zrow_v, rows_v, out_s, sem_g):
    s = lax.axis_index("s")

    zero16 = jnp.zeros((16,), jnp.float32)
    for r in range(_ZR):
        for k in range(_D // 16):
            zrow_v[r, pl.ds(k * 16, 16)] = zero16
    for i in range(_NPT // _ZR):
        pltpu.sync_copy(zrow_v, out_s.at[pl.ds(s * _NPT + i * _ZR, _ZR)])
    plsc.subcore_barrier()

    def main_blk(sb, carry):
        pltpu.sync_copy(gidx_hbm.at[s, sb], gblk_v)
        pltpu.sync_copy(sidx_hbm.at[s, sb], sblk_v)
        pltpu.sync_copy(c_hbm.at[s, sb], cblk_v)

        descs = [None] * _SB
        descs[0] = pltpu.async_copy(h_hbm.at[gblk_v.at[0]], rows_v.at[0],
                                    sem_g)
        for j in range(_SB):
            buf = j % 2
            descs[j].wait()
            if j + 1 < _SB:
                descs[j + 1] = pltpu.async_copy(
                    h_hbm.at[gblk_v.at[j + 1]], rows_v.at[(j + 1) % 2], sem_g)

            def grp_body(g, carry3, j=j, buf=buf):
                cvec = cblk_v[j, pl.ds(g * 16, 16)]
                for el in range(16):
                    cb = jnp.full((16,), cvec[el], jnp.float32)
                    e = g * 16 + el
                    for k in range(_D // 16):
                        rows_v[buf, e, pl.ds(k * 16, 16)] = (
                            rows_v[buf, e, pl.ds(k * 16, 16)] * cb)
                return carry3
            lax.fori_loop(0, _CH // 16, grp_body, 0)

            pltpu.sync_copy(rows_v.at[buf], out_s.at[sblk_v.at[j]], add=True)
        return carry
    lax.fori_loop(0, _NSB, main_blk, 0)
    plsc.subcore_barrier()

    pltpu.sync_copy(out_s.at[pl.ds(s * _NPT, _NPT)], out_hbm.at[s])


def _comb_body(p_ref, b_ref, o_ref):
    o_ref[...] = p_ref[:_N] + b_ref[...]


def _combine(partial, bias2d):
    return pl.pallas_call(
        _comb_body,
        out_shape=jax.ShapeDtypeStruct((_N, _D), jnp.float32),
    )(partial, bias2d)


def kernel(node_embeddings, triples, basis, att, bias):
    src = triples[:, 0].astype(jnp.int32)
    rel = triples[:, 1].astype(jnp.int32)
    dst = triples[:, 2].astype(jnp.int32)
    gidx = (rel * _N + dst).reshape(_NS, _NSB, _SB, _CH)
    didx = (rel * _N + src).reshape(_NS, _NSB, _SB, _CH)
    sidx = src.reshape(_NS, _NSB, _SB, _CH)
    cpe = _sc_degc(didx)
    h = _build_h(node_embeddings, basis, att).reshape(_RN, _D)
    partial = _sc_aggregate(h, gidx, sidx, cpe)
    partial = partial.reshape(_NPAD, _D)
    return _combine(partial, bias.reshape(1, _D))


# speedup vs baseline: 18.6005x; 1.2379x over previous
import functools

import jax
import jax.numpy as jnp
from jax import lax
from jax.experimental import pallas as pl
from jax.experimental.pallas import tpu as pltpu
from jax.experimental.pallas import tpu_sc as plsc

_N = 10000
_R = 16
_B = 8
_D = 128
_E = 320000

_NS = 16

_CH = 80
_W_EDGES = _E // _NS
_W_ROWS = _W_EDGES // _CH
_SB = 25
_NSB = _W_ROWS // _SB
_RN = _R * _N
_NPAD = 10112
_NPT = _NPAD // _NS
_ZR = 8
_ZD = 1000
_DW = 8


def _h_body(att_ref, basis_ref, emb_ref, h_ref):
    r = pl.program_id(0)
    w = att_ref[r, 0] * basis_ref[0]
    for b in range(1, _B):
        w = w + att_ref[r, b] * basis_ref[b]
    h_ref[0] = jnp.dot(emb_ref[...], w, preferred_element_type=jnp.float32)


def _build_h(emb, basis, att):
    return pl.pallas_call(
        _h_body,
        grid=(_R,),
        in_specs=[
            pl.BlockSpec((_R, _B), lambda r: (0, 0)),
            pl.BlockSpec((_B, _D, _D), lambda r: (0, 0, 0)),
            pl.BlockSpec((_N, _D), lambda r: (0, 0)),
        ],
        out_specs=pl.BlockSpec((1, _N, _D), lambda r: (r, 0, 0)),
        out_shape=jax.ShapeDtypeStruct((_R, _N, _D), jnp.float32),
    )(att, basis, emb)


_mesh = plsc.VectorSubcoreMesh(core_axis_name="c", subcore_axis_name="s",
                               num_cores=1)


@functools.partial(
    pl.kernel,
    out_type=jax.ShapeDtypeStruct((_NS, _NSB, _SB, _CH), jnp.float32),
    mesh=_mesh,
    scratch_types=[
        pltpu.VMEM((_SB, _CH), jnp.int32),
        pltpu.VMEM((_SB, _CH), jnp.float32),
        pltpu.VMEM((2, _CH), jnp.float32),
        pltpu.VMEM((_CH,), jnp.float32),
        pltpu.VMEM((_ZD,), jnp.float32),
        pltpu.VMEM_SHARED((_RN,), jnp.float32),
        pltpu.SemaphoreType.DMA,
        pltpu.SemaphoreType.DMA,
    ],
)
def _sc_degc(didx_hbm, c_hbm, dblk_v, cblk_v, dtmp_v, ones_v, zdeg_v,
             deg_s, sem_d, sem_c):
    s = lax.axis_index("s")

    zero16 = jnp.zeros((16,), jnp.float32)
    one16 = jnp.ones((16,), jnp.float32)
    for k in range(_ZD // 16):
        zdeg_v[pl.ds(k * 16, 16)] = zero16
    for k in range(_CH // 16):
        ones_v[pl.ds(k * 16, 16)] = one16

    for i in range(_RN // _NS // _ZD):
        pltpu.sync_copy(zdeg_v, deg_s.at[pl.ds(s * (_RN // _NS) + i * _ZD, _ZD)])
    plsc.subcore_barrier()

    def deg_blk(sb, carry):
        pltpu.sync_copy(didx_hbm.at[s, sb], dblk_v)
        descs = [None] * _SB
        for j in range(_SB):
            descs[j] = pltpu.async_copy(ones_v, deg_s.at[dblk_v.at[j]],
                                        sem_d, add=True)
            if j >= _DW:
                descs[j - _DW].wait()
        for j in range(_SB - _DW, _SB):
            descs[j].wait()
        return carry
    lax.fori_loop(0, _NSB, deg_blk, 0)
    plsc.subcore_barrier()

    def c_blk(sb, carry):
        pltpu.sync_copy(didx_hbm.at[s, sb], dblk_v)
        descs = [None] * _SB
        descs[0] = pltpu.async_copy(deg_s.at[dblk_v.at[0]], dtmp_v.at[0],
                                    sem_c)
        for j in range(_SB):
            descs[j].wait()
            if j + 1 < _SB:
                descs[j + 1] = pltpu.async_copy(
                    deg_s.at[dblk_v.at[j + 1]], dtmp_v.at[(j + 1) % 2], sem_c)
            for k in range(_CH // 16):
                cblk_v[j, pl.ds(k * 16, 16)] = (
                    1.0 / dtmp_v[j % 2, pl.ds(k * 16, 16)])
        pltpu.sync_copy(cblk_v, c_hbm.at[s, sb])
        return carry
    lax.fori_loop(0, _NSB, c_blk, 0)


@functools.partial(
    pl.kernel,
    out_type=jax.ShapeDtypeStruct((_NS, _NPT, _D), jnp.float32),
    mesh=_mesh,
    scratch_types=[
        pltpu.VMEM((_SB, _CH), jnp.int32),
        pltpu.VMEM((_SB, _CH), jnp.int32),
        pltpu.VMEM((_SB, _CH), jnp.float32),
        pltpu.VMEM((_ZR, _D), jnp.float32),
        pltpu.VMEM((3, _CH, _D), jnp.float32),
        pltpu.VMEM_SHARED((_NPAD, _D), jnp.float32),
        pltpu.SemaphoreType.DMA,
        pltpu.SemaphoreType.DMA,
    ],
)
def _sc_aggregate(h_hbm, gidx_hbm, sidx_hbm, c_hbm, out_hbm,
                  gblk_v, sblk_v, cblk_v, zrow_v, rows_v, out_s,
                  sem_g, sem_s):
    s = lax.axis_index("s")

    zero16 = jnp.zeros((16,), jnp.float32)
    for r in range(_ZR):
        for k in range(_D // 16):
            zrow_v[r, pl.ds(k * 16, 16)] = zero16
    for i in range(_NPT // _ZR):
        pltpu.sync_copy(zrow_v, out_s.at[pl.ds(s * _NPT + i * _ZR, _ZR)])
    plsc.subcore_barrier()

    def main_blk(sb, carry):
        pltpu.sync_copy(gidx_hbm.at[s, sb], gblk_v)
        pltpu.sync_copy(sidx_hbm.at[s, sb], sblk_v)
        pltpu.sync_copy(c_hbm.at[s, sb], cblk_v)

        g_descs = [None] * _SB
        s_descs = [None] * _SB
        g_descs[0] = pltpu.async_copy(h_hbm.at[gblk_v.at[0]], rows_v.at[0],
                                      sem_g)
        g_descs[1] = pltpu.async_copy(h_hbm.at[gblk_v.at[1]], rows_v.at[1],
                                      sem_g)
        for j in range(_SB):
            buf = j % 3
            g_descs[j].wait()

            def grp_body(g, carry3, j=j, buf=buf):
                cvec = cblk_v[j, pl.ds(g * 16, 16)]
                for el in range(16):
                    cb = jnp.full((16,), cvec[el], jnp.float32)
                    e = g * 16 + el
                    for k in range(_D // 16):
                        rows_v[buf, e, pl.ds(k * 16, 16)] = (
                            rows_v[buf, e, pl.ds(k * 16, 16)] * cb)
                return carry3
            lax.fori_loop(0, _CH // 16, grp_body, 0)

            s_descs[j] = pltpu.async_copy(rows_v.at[buf],
                                          out_s.at[sblk_v.at[j]], sem_s,
                                          add=True)
            if j + 2 < _SB:
                if j >= 1:
                    s_descs[j - 1].wait()
                g_descs[j + 2] = pltpu.async_copy(
                    h_hbm.at[gblk_v.at[j + 2]], rows_v.at[(j + 2) % 3], sem_g)
        for j in range(_SB - 3, _SB):
            s_descs[j].wait()
        return carry
    lax.fori_loop(0, _NSB, main_blk, 0)
    plsc.subcore_barrier()

    pltpu.sync_copy(out_s.at[pl.ds(s * _NPT, _NPT)], out_hbm.at[s])


def _comb_body(p_ref, b_ref, o_ref):
    o_ref[...] = p_ref[:_N] + b_ref[...]


def _combine(partial, bias2d):
    return pl.pallas_call(
        _comb_body,
        out_shape=jax.ShapeDtypeStruct((_N, _D), jnp.float32),
    )(partial, bias2d)


def kernel(node_embeddings, triples, basis, att, bias):
    src = triples[:, 0].astype(jnp.int32)
    rel = triples[:, 1].astype(jnp.int32)
    dst = triples[:, 2].astype(jnp.int32)
    gidx = (rel * _N + dst).reshape(_NS, _NSB, _SB, _CH)
    didx = (rel * _N + src).reshape(_NS, _NSB, _SB, _CH)
    sidx = src.reshape(_NS, _NSB, _SB, _CH)
    cpe = _sc_degc(didx)
    h = _build_h(node_embeddings, basis, att).reshape(_RN, _D)
    partial = _sc_aggregate(h, gidx, sidx, cpe)
    partial = partial.reshape(_NPAD, _D)
    return _combine(partial, bias.reshape(1, _D))


# speedup vs baseline: 19.4568x; 1.0460x over previous
import functools

import jax
import jax.numpy as jnp
from jax import lax
from jax.experimental import pallas as pl
from jax.experimental.pallas import tpu as pltpu
from jax.experimental.pallas import tpu_sc as plsc

_N = 10000
_R = 16
_B = 8
_D = 128
_E = 320000

_NS = 16

_CH = 80
_W_EDGES = _E // _NS
_W_ROWS = _W_EDGES // _CH
_SB = 25
_NSB = _W_ROWS // _SB
_RN = _R * _N
_NPAD = 10112
_NPT = _NPAD // _NS
_ZR = 8
_ZD = 1000
_DW = 12


def _h_body(att_ref, basis_ref, emb_ref, h_ref):
    r = pl.program_id(0)
    w = att_ref[r, 0] * basis_ref[0]
    for b in range(1, _B):
        w = w + att_ref[r, b] * basis_ref[b]
    h_ref[0] = jnp.dot(emb_ref[...], w, preferred_element_type=jnp.float32)


def _build_h(emb, basis, att):
    return pl.pallas_call(
        _h_body,
        grid=(_R,),
        in_specs=[
            pl.BlockSpec((_R, _B), lambda r: (0, 0)),
            pl.BlockSpec((_B, _D, _D), lambda r: (0, 0, 0)),
            pl.BlockSpec((_N, _D), lambda r: (0, 0)),
        ],
        out_specs=pl.BlockSpec((1, _N, _D), lambda r: (r, 0, 0)),
        out_shape=jax.ShapeDtypeStruct((_R, _N, _D), jnp.float32),
    )(att, basis, emb)


_mesh = plsc.VectorSubcoreMesh(core_axis_name="c", subcore_axis_name="s",
                               num_cores=1)


@functools.partial(
    pl.kernel,
    out_type=jax.ShapeDtypeStruct((_NS, _NSB, _SB, _CH), jnp.float32),
    mesh=_mesh,
    scratch_types=[
        pltpu.VMEM((_SB, _CH), jnp.int32),
        pltpu.VMEM((_SB, _CH), jnp.float32),
        pltpu.VMEM((4, _CH), jnp.float32),
        pltpu.VMEM((_CH,), jnp.float32),
        pltpu.VMEM((_ZD,), jnp.float32),
        pltpu.VMEM_SHARED((_RN,), jnp.float32),
        pltpu.SemaphoreType.DMA,
        pltpu.SemaphoreType.DMA,
    ],
)
def _sc_degc(didx_hbm, c_hbm, dblk_v, cblk_v, dtmp_v, ones_v, zdeg_v,
             deg_s, sem_d, sem_c):
    s = lax.axis_index("s")

    zero16 = jnp.zeros((16,), jnp.float32)
    one16 = jnp.ones((16,), jnp.float32)
    for k in range(_ZD // 16):
        zdeg_v[pl.ds(k * 16, 16)] = zero16
    for k in range(_CH // 16):
        ones_v[pl.ds(k * 16, 16)] = one16

    for i in range(_RN // _NS // _ZD):
        pltpu.sync_copy(zdeg_v, deg_s.at[pl.ds(s * (_RN // _NS) + i * _ZD, _ZD)])
    plsc.subcore_barrier()

    def deg_blk(sb, carry):
        pltpu.sync_copy(didx_hbm.at[s, sb], dblk_v)
        descs = [None] * _SB
        for j in range(_SB):
            descs[j] = pltpu.async_copy(ones_v, deg_s.at[dblk_v.at[j]],
                                        sem_d, add=True)
            if j >= _DW:
                descs[j - _DW].wait()
        for j in range(_SB - _DW, _SB):
            descs[j].wait()
        return carry
    lax.fori_loop(0, _NSB, deg_blk, 0)
    plsc.subcore_barrier()

    def c_blk(sb, carry):
        pltpu.sync_copy(didx_hbm.at[s, sb], dblk_v)
        descs = [None] * _SB
        for p in range(3):
            descs[p] = pltpu.async_copy(deg_s.at[dblk_v.at[p]], dtmp_v.at[p],
                                        sem_c)
        for j in range(_SB):
            descs[j].wait()
            if j + 3 < _SB:
                descs[j + 3] = pltpu.async_copy(
                    deg_s.at[dblk_v.at[j + 3]], dtmp_v.at[(j + 3) % 4], sem_c)
            for k in range(_CH // 16):
                cblk_v[j, pl.ds(k * 16, 16)] = (
                    1.0 / dtmp_v[j % 4, pl.ds(k * 16, 16)])
        pltpu.sync_copy(cblk_v, c_hbm.at[s, sb])
        return carry
    lax.fori_loop(0, _NSB, c_blk, 0)


@functools.partial(
    pl.kernel,
    out_type=jax.ShapeDtypeStruct((_NS, _NPT, _D), jnp.float32),
    mesh=_mesh,
    scratch_types=[
        pltpu.VMEM((_SB, _CH), jnp.int32),
        pltpu.VMEM((_SB, _CH), jnp.int32),
        pltpu.VMEM((_SB, _CH), jnp.float32),
        pltpu.VMEM((1, _D), jnp.float32),
        pltpu.VMEM((_ZR, _D), jnp.float32),
        pltpu.VMEM((3, _CH, _D), jnp.float32),
        pltpu.VMEM_SHARED((_NPAD, _D), jnp.float32),
        pltpu.SemaphoreType.DMA,
        pltpu.SemaphoreType.DMA,
    ],
)
def _sc_aggregate(h_hbm, gidx_hbm, sidx_hbm, c_hbm, bias_hbm, out_hbm,
                  gblk_v, sblk_v, cblk_v, biasb_v, zrow_v, rows_v, out_s,
                  sem_g, sem_s):
    s = lax.axis_index("s")

    pltpu.sync_copy(bias_hbm, biasb_v)
    for r in range(_ZR):
        for k in range(_D // 16):
            zrow_v[r, pl.ds(k * 16, 16)] = biasb_v[0, pl.ds(k * 16, 16)]
    for i in range(_NPT // _ZR):
        pltpu.sync_copy(zrow_v, out_s.at[pl.ds(s * _NPT + i * _ZR, _ZR)])
    plsc.subcore_barrier()

    def main_blk(sb, carry):
        pltpu.sync_copy(gidx_hbm.at[s, sb], gblk_v)
        pltpu.sync_copy(sidx_hbm.at[s, sb], sblk_v)
        pltpu.sync_copy(c_hbm.at[s, sb], cblk_v)

        g_descs = [None] * _SB
        s_descs = [None] * _SB
        for p in range(2):
            g_descs[p] = pltpu.async_copy(h_hbm.at[gblk_v.at[p]],
                                          rows_v.at[p], sem_g)
        for j in range(_SB):
            buf = j % 3
            g_descs[j].wait()

            def grp_body(g, carry3, j=j, buf=buf):
                cvec = cblk_v[j, pl.ds(g * 16, 16)]
                for el in range(16):
                    cb = jnp.full((16,), cvec[el], jnp.float32)
                    e = g * 16 + el
                    for k in range(_D // 16):
                        rows_v[buf, e, pl.ds(k * 16, 16)] = (
                            rows_v[buf, e, pl.ds(k * 16, 16)] * cb)
                return carry3
            lax.fori_loop(0, _CH // 16, grp_body, 0)

            s_descs[j] = pltpu.async_copy(rows_v.at[buf],
                                          out_s.at[sblk_v.at[j]], sem_s,
                                          add=True)
            if j + 2 < _SB:
                if j >= 1:
                    s_descs[j - 1].wait()
                g_descs[j + 2] = pltpu.async_copy(
                    h_hbm.at[gblk_v.at[j + 2]], rows_v.at[(j + 2) % 3], sem_g)
        for j in range(_SB - 3, _SB):
            s_descs[j].wait()
        return carry
    lax.fori_loop(0, _NSB, main_blk, 0)
    plsc.subcore_barrier()

    pltpu.sync_copy(out_s.at[pl.ds(s * _NPT, _NPT)], out_hbm.at[s])


def kernel(node_embeddings, triples, basis, att, bias):
    src = triples[:, 0].astype(jnp.int32)
    rel = triples[:, 1].astype(jnp.int32)
    dst = triples[:, 2].astype(jnp.int32)
    gidx = (rel * _N + dst).reshape(_NS, _NSB, _SB, _CH)
    didx = (rel * _N + src).reshape(_NS, _NSB, _SB, _CH)
    sidx = src.reshape(_NS, _NSB, _SB, _CH)
    cpe = _sc_degc(didx)
    h = _build_h(node_embeddings, basis, att).reshape(_RN, _D)
    partial = _sc_aggregate(h, gidx, sidx, cpe, bias.reshape(1, _D))
    return partial.reshape(_NPAD, _D)[:_N]
